# Initial kernel scaffold; baseline (speedup 1.0000x reference)
import jax
import jax.numpy as jnp
from jax.experimental import pallas as pl


def kernel(x, W_enc, b_enc, W_dec, b_dec, centroids):
    raise NotImplementedError("write your pallas kernel here")


# speedup vs baseline: 1.0081x; 1.0081x over previous
import functools

import jax
import jax.numpy as jnp
from jax import lax
from jax.experimental import pallas as pl
from jax.experimental.pallas import tpu as pltpu
from jax.experimental.pallas import tpu_sc as plsc

N = 8192
K = 8192
D = 32
C = 96
CP = 128
BN = 1024
BK = 2048
NT = N // BN
KT = K // BK

_bf16 = jnp.bfloat16


def _snap_bf16(v):
    u = lax.bitcast_convert_type(v, jnp.uint32)
    r = (u + ((u >> 16) & 1) + jnp.uint32(0x7FFF)) & jnp.uint32(0xFFFF0000)
    return lax.bitcast_convert_type(r, jnp.float32)


def _vq_tc_kernel(hb_ref, xcol_ref, yrow_ref, ctb_ref, centsb_ref, wdb_ref,
                  bd_ref, dec_ref, idx_ref, rmin_s, ridx_s):
    n = pl.program_id(0)
    k = pl.program_id(1)

    @pl.when((n == 0) & (k == 0))
    def _prologue():
        dec_ref[...] = lax.dot_general(
            centsb_ref[...], wdb_ref[...], (((1,), (0,)), ((), ())),
            preferred_element_type=jnp.float32) + bd_ref[...]

    @pl.when(k == 0)
    def _init_row():
        rmin_s[...] = jnp.full((BN, 1), jnp.inf, jnp.float32)
        ridx_s[...] = jnp.zeros((BN, 1), jnp.int32)

    hb = hb_ref[pl.ds(n * BN, BN), :]
    ctb = ctb_ref[:, pl.ds(k * BK, BK)]
    s = lax.dot_general(hb, ctb, (((1,), (0,)), ((), ())),
                        preferred_element_type=jnp.float32)
    dist = (-2.0 * s + xcol_ref[pl.ds(n * BN, BN), :]) \
        + yrow_ref[:, pl.ds(k * BK, BK)]
    tmin = jnp.min(dist, axis=1, keepdims=True)
    cols = lax.broadcasted_iota(jnp.int32, (BN, BK), 1)
    targ = jnp.min(jnp.where(dist == tmin, cols, K), axis=1,
                   keepdims=True) + k * BK
    prev_min = rmin_s[...]
    upd = tmin < prev_min
    rmin_s[...] = jnp.where(upd, _snap_bf16(tmin), prev_min)
    ridx_s[...] = jnp.where(upd, targ, ridx_s[...])

    @pl.when(k == KT - 1)
    def _flush_row():
        idx_ref[pl.ds(n * BN, BN), :] = ridx_s[...]


def _vq_tc(hb, xcol, yrow, ctb, centsb, wdb, bd2):
    full = lambda shape: pl.BlockSpec(shape, lambda n, k: (0, 0))
    return pl.pallas_call(
        _vq_tc_kernel,
        grid=(NT, KT),
        in_specs=[
            full((N, D)), full((N, 1)), full((1, K)), full((D, K)),
            full((K, D)), full((D, CP)), full((1, CP)),
        ],
        out_specs=[full((K, CP)), full((N, 1))],
        out_shape=[
            jax.ShapeDtypeStruct((K, CP), jnp.float32),
            jax.ShapeDtypeStruct((N, 1), jnp.int32),
        ],
        scratch_shapes=[
            pltpu.VMEM((BN, 1), jnp.float32),
            pltpu.VMEM((BN, 1), jnp.int32),
        ],
    )(hb, xcol, yrow, ctb, centsb, wdb, bd2)


NC, NS = 2, 16
NW = NC * NS
BPW = N // NW
CH = 128
NCH = BPW // CH


def _sc_gather(decoded, idx3):
    mesh = plsc.VectorSubcoreMesh(core_axis_name="c", subcore_axis_name="s")

    @functools.partial(
        pl.kernel, mesh=mesh,
        out_type=jax.ShapeDtypeStruct((N, CP), jnp.float32),
        scratch_types=[
            pltpu.VMEM((NCH, CH), jnp.int32),
            pltpu.VMEM((BPW, CP), jnp.float32),
            pltpu.SemaphoreType.DMA,
        ],
    )
    def gather_k(dec_hbm, idx_hbm, out_hbm, idx_v, rows_v, sem):
        wid = lax.axis_index("s") * NC + lax.axis_index("c")
        pltpu.sync_copy(idx_hbm.at[wid], idx_v)
        for j in range(NCH):
            pltpu.async_copy(dec_hbm.at[idx_v.at[j]],
                             rows_v.at[pl.ds(j * CH, CH)], sem)
        for j in range(NCH):
            pltpu.make_async_copy(dec_hbm.at[idx_v.at[j]],
                                  rows_v.at[pl.ds(j * CH, CH)], sem).wait()
        pltpu.sync_copy(rows_v, out_hbm.at[pl.ds(wid * BPW, BPW)])

    return gather_k(decoded, idx3)


def kernel(x, W_enc, b_enc, W_dec, b_dec, centroids):
    h = x @ W_enc + b_enc
    flat = h.reshape(-1, h.shape[-1])
    xcol = jnp.sum(flat * flat, axis=1)[:, None]
    yrow = jnp.sum(centroids * centroids, axis=1)[None, :]

    W_dec_p = jnp.pad(W_dec, ((0, 0), (0, CP - C)))
    b_dec_p = jnp.pad(b_dec, (0, CP - C)).reshape(1, CP)
    decoded, idx = _vq_tc(flat.astype(_bf16), xcol, yrow,
                          centroids.T.astype(_bf16), centroids.astype(_bf16),
                          W_dec_p.astype(_bf16), b_dec_p)
    idx3 = idx.reshape(NW, NCH, CH)
    out = _sc_gather(decoded, idx3)
    return out[:, :C].reshape(x.shape[0], x.shape[1], C)


# speedup vs baseline: 1.1544x; 1.1451x over previous
import functools

import jax
import jax.numpy as jnp
from jax import lax
from jax.experimental import pallas as pl
from jax.experimental.pallas import tpu as pltpu
from jax.experimental.pallas import tpu_sc as plsc

N = 8192
K = 8192
D = 32
C = 96
CP = 128
BN = 1024
BK = 2048
NT = N // BN
KT = K // BK

_bf16 = jnp.bfloat16


def _snap_bf16(v):
    u = lax.bitcast_convert_type(v, jnp.uint32)
    r = (u + ((u >> 16) & 1) + jnp.uint32(0x7FFF)) & jnp.uint32(0xFFFF0000)
    return lax.bitcast_convert_type(r, jnp.float32)


def _vq_tc_kernel(hb_ref, xcol_ref, yrow_ref, ctb_ref, centsb_ref, wdb_ref,
                  bd_ref, dec_ref, idx_ref, rmin_s, ridx_s):
    n = pl.program_id(0)
    k = pl.program_id(1)

    @pl.when((n == 0) & (k == 0))
    def _prologue():
        dec_ref[...] = lax.dot_general(
            centsb_ref[...], wdb_ref[...], (((1,), (0,)), ((), ())),
            preferred_element_type=jnp.float32) + bd_ref[...]

    @pl.when(k == 0)
    def _init_row():
        rmin_s[...] = jnp.full((BN, 1), jnp.inf, jnp.float32)
        ridx_s[...] = jnp.zeros((BN, 1), jnp.int32)

    hb2 = hb_ref[pl.ds(n * BN, BN), :]
    ctb = ctb_ref[:, pl.ds(k * BK, BK)]
    s2 = lax.dot_general(hb2, ctb, (((1,), (0,)), ((), ())),
                         preferred_element_type=jnp.float32)
    dist = (s2 + xcol_ref[pl.ds(n * BN, BN), :]) \
        + yrow_ref[:, pl.ds(k * BK, BK)]
    LT = BK // 128
    m = dist[:, 0:128]
    tt = jnp.zeros((BN, 128), jnp.int32)
    for t in range(1, LT):
        dt = dist[:, t * 128:(t + 1) * 128]
        lt = dt < m
        m = jnp.where(lt, dt, m)
        tt = jnp.where(lt, t, tt)
    cm = jnp.min(m, axis=1, keepdims=True)
    lanes = lax.broadcasted_iota(jnp.int32, (BN, 128), 1)
    jj = tt * 128 + lanes
    targ = jnp.min(jnp.where(m == cm, jj, K), axis=1,
                   keepdims=True) + k * BK
    prev_min = rmin_s[...]
    upd = cm < prev_min
    rmin_s[...] = jnp.where(upd, _snap_bf16(cm), prev_min)
    ridx_s[...] = jnp.where(upd, targ, ridx_s[...])

    @pl.when(k == KT - 1)
    def _flush_row():
        idx_ref[pl.ds(n * BN, BN), :] = ridx_s[...]


def _vq_tc(hb, xcol, yrow, ctb, centsb, wdb, bd2):
    full = lambda shape: pl.BlockSpec(shape, lambda n, k: (0, 0))
    return pl.pallas_call(
        _vq_tc_kernel,
        grid=(NT, KT),
        in_specs=[
            full((N, D)), full((N, 1)), full((1, K)), full((D, K)),
            full((K, D)), full((D, CP)), full((1, CP)),
        ],
        out_specs=[full((K, CP)), full((N, 1))],
        out_shape=[
            jax.ShapeDtypeStruct((K, CP), jnp.float32),
            jax.ShapeDtypeStruct((N, 1), jnp.int32),
        ],
        scratch_shapes=[
            pltpu.VMEM((BN, 1), jnp.float32),
            pltpu.VMEM((BN, 1), jnp.int32),
        ],
    )(hb, xcol, yrow, ctb, centsb, wdb, bd2)


NC, NS = 2, 16
NW = NC * NS
BPW = N // NW
CH = 128
NCH = BPW // CH


def _sc_gather(decoded, idx3):
    mesh = plsc.VectorSubcoreMesh(core_axis_name="c", subcore_axis_name="s")

    @functools.partial(
        pl.kernel, mesh=mesh,
        out_type=jax.ShapeDtypeStruct((N, CP), jnp.float32),
        scratch_types=[
            pltpu.VMEM((NCH, CH), jnp.int32),
            pltpu.VMEM((BPW, CP), jnp.float32),
            pltpu.SemaphoreType.DMA,
        ],
    )
    def gather_k(dec_hbm, idx_hbm, out_hbm, idx_v, rows_v, sem):
        wid = lax.axis_index("s") * NC + lax.axis_index("c")
        pltpu.sync_copy(idx_hbm.at[wid], idx_v)
        for j in range(NCH):
            pltpu.async_copy(dec_hbm.at[idx_v.at[j]],
                             rows_v.at[pl.ds(j * CH, CH)], sem)
        for j in range(NCH):
            pltpu.make_async_copy(dec_hbm.at[idx_v.at[j]],
                                  rows_v.at[pl.ds(j * CH, CH)], sem).wait()
        pltpu.sync_copy(rows_v, out_hbm.at[pl.ds(wid * BPW, BPW)])

    return gather_k(decoded, idx3)


def kernel(x, W_enc, b_enc, W_dec, b_dec, centroids):
    h = x @ W_enc + b_enc
    flat = h.reshape(-1, h.shape[-1])
    xcol = jnp.sum(flat * flat, axis=1)[:, None]
    yrow = jnp.sum(centroids * centroids, axis=1)[None, :]

    W_dec_p = jnp.pad(W_dec, ((0, 0), (0, CP - C)))
    b_dec_p = jnp.pad(b_dec, (0, CP - C)).reshape(1, CP)
    decoded, idx = _vq_tc(flat.astype(_bf16) * _bf16(-2.0), xcol, yrow,
                          centroids.T.astype(_bf16), centroids.astype(_bf16),
                          W_dec_p.astype(_bf16), b_dec_p)
    idx3 = idx.reshape(NW, NCH, CH)
    out = _sc_gather(decoded, idx3)
    return out[:, :C].reshape(x.shape[0], x.shape[1], C)


# speedup vs baseline: 1.1592x; 1.0042x over previous
import functools

import jax
import jax.numpy as jnp
from jax import lax
from jax.experimental import pallas as pl
from jax.experimental.pallas import tpu as pltpu
from jax.experimental.pallas import tpu_sc as plsc

N = 8192
K = 8192
D = 32
C = 96
CP = 128
BN = 1024
BK = 2048
NT = N // BN
KT = K // BK

_bf16 = jnp.bfloat16


def _snap_bf16(v):
    u = lax.bitcast_convert_type(v, jnp.uint32)
    r = (u + ((u >> 16) & 1) + jnp.uint32(0x7FFF)) & jnp.uint32(0xFFFF0000)
    return lax.bitcast_convert_type(r, jnp.float32)


def _vq_tc_kernel(hb_ref, xcol_ref, yrow_ref, ctb_ref, centsb_ref, wdb_ref,
                  bd_ref, dec_ref, idx_ref, rmin_s, ridx_s):
    n = pl.program_id(0)
    k = pl.program_id(1)

    @pl.when((n == 0) & (k == 0))
    def _prologue():
        dec_ref[...] = lax.dot_general(
            centsb_ref[...], wdb_ref[...], (((1,), (0,)), ((), ())),
            preferred_element_type=jnp.float32) + bd_ref[...]

    @pl.when(k == 0)
    def _init_row():
        rmin_s[...] = jnp.full((BN, 1), jnp.inf, jnp.float32)
        ridx_s[...] = jnp.zeros((BN, 1), jnp.int32)

    hb2 = hb_ref[pl.ds(n * BN, BN), :]
    ctb = ctb_ref[:, pl.ds(k * BK, BK)]
    s2 = lax.dot_general(hb2, ctb, (((1,), (0,)), ((), ())),
                         preferred_element_type=jnp.float32)
    xn = xcol_ref[pl.ds(n * BN, BN), :]
    LT = BK // 128
    m = (s2[:, 0:128] + xn) + yrow_ref[:, pl.ds(k * BK, 128)]
    tt = jnp.zeros((BN, 128), jnp.int32)
    for t in range(1, LT):
        dt = (s2[:, t * 128:(t + 1) * 128] + xn) \
            + yrow_ref[:, pl.ds(k * BK + t * 128, 128)]
        lt = dt < m
        m = jnp.where(lt, dt, m)
        tt = jnp.where(lt, t, tt)
    cm = jnp.min(m, axis=1, keepdims=True)
    lanes = lax.broadcasted_iota(jnp.int32, (BN, 128), 1)
    jj = tt * 128 + lanes
    targ = jnp.min(jnp.where(m == cm, jj, K), axis=1,
                   keepdims=True) + k * BK
    prev_min = rmin_s[...]
    upd = cm < prev_min
    rmin_s[...] = jnp.where(upd, _snap_bf16(cm), prev_min)
    ridx_s[...] = jnp.where(upd, targ, ridx_s[...])

    @pl.when(k == KT - 1)
    def _flush_row():
        idx_ref[pl.ds(n * BN, BN), :] = ridx_s[...]


def _vq_tc(hb, xcol, yrow, ctb, centsb, wdb, bd2):
    full = lambda shape: pl.BlockSpec(shape, lambda n, k: (0, 0))
    return pl.pallas_call(
        _vq_tc_kernel,
        grid=(NT, KT),
        in_specs=[
            full((N, D)), full((N, 1)), full((1, K)), full((D, K)),
            full((K, D)), full((D, CP)), full((1, CP)),
        ],
        out_specs=[full((K, CP)), full((N, 1))],
        out_shape=[
            jax.ShapeDtypeStruct((K, CP), jnp.float32),
            jax.ShapeDtypeStruct((N, 1), jnp.int32),
        ],
        scratch_shapes=[
            pltpu.VMEM((BN, 1), jnp.float32),
            pltpu.VMEM((BN, 1), jnp.int32),
        ],
    )(hb, xcol, yrow, ctb, centsb, wdb, bd2)


NC, NS = 2, 16
NW = NC * NS
BPW = N // NW
CH = 128
NCH = BPW // CH


def _sc_gather(decoded, idx3):
    mesh = plsc.VectorSubcoreMesh(core_axis_name="c", subcore_axis_name="s")

    @functools.partial(
        pl.kernel, mesh=mesh,
        out_type=jax.ShapeDtypeStruct((N, CP), jnp.float32),
        scratch_types=[
            pltpu.VMEM((NCH, CH), jnp.int32),
            pltpu.VMEM((BPW, CP), jnp.float32),
            pltpu.SemaphoreType.DMA,
        ],
    )
    def gather_k(dec_hbm, idx_hbm, out_hbm, idx_v, rows_v, sem):
        wid = lax.axis_index("s") * NC + lax.axis_index("c")
        pltpu.sync_copy(idx_hbm.at[wid], idx_v)
        for j in range(NCH):
            pltpu.async_copy(dec_hbm.at[idx_v.at[j]],
                             rows_v.at[pl.ds(j * CH, CH)], sem)
        for j in range(NCH):
            pltpu.make_async_copy(dec_hbm.at[idx_v.at[j]],
                                  rows_v.at[pl.ds(j * CH, CH)], sem).wait()
        pltpu.sync_copy(rows_v, out_hbm.at[pl.ds(wid * BPW, BPW)])

    return gather_k(decoded, idx3)


def kernel(x, W_enc, b_enc, W_dec, b_dec, centroids):
    h = x @ W_enc + b_enc
    flat = h.reshape(-1, h.shape[-1])
    xcol = jnp.sum(flat * flat, axis=1)[:, None]
    yrow = jnp.sum(centroids * centroids, axis=1)[None, :]

    W_dec_p = jnp.pad(W_dec, ((0, 0), (0, CP - C)))
    b_dec_p = jnp.pad(b_dec, (0, CP - C)).reshape(1, CP)
    decoded, idx = _vq_tc(flat.astype(_bf16) * _bf16(-2.0), xcol, yrow,
                          centroids.T.astype(_bf16), centroids.astype(_bf16),
                          W_dec_p.astype(_bf16), b_dec_p)
    idx3 = idx.reshape(NW, NCH, CH)
    out = _sc_gather(decoded, idx3)
    return out[:, :C].reshape(x.shape[0], x.shape[1], C)
